# baseline (speedup 1.0000x reference)
import functools

import jax
import jax.numpy as jnp
from jax import lax
from jax.experimental import pallas as pl
from jax.experimental.pallas import tpu as pltpu
from jax.experimental.pallas import tpu_sc as plsc

_EPS = 1e-5
_NC = 2
_NS = 16
_CHUNK = 128
_BM = 512
_NBUF = 2
_NPH = 2
_IBLK = 8


def _round_up(a: int, b: int) -> int:
    return (a + b - 1) // b * b


def _make_deg_kernel(nchunks: int, np_rows: int):
    mesh = plsc.VectorSubcoreMesh(core_axis_name="c", subcore_axis_name="s")
    nt = nchunks // _NS
    rows_sub = np_rows // _NS

    @functools.partial(
        pl.kernel,
        out_type=jax.ShapeDtypeStruct((_NC * np_rows,), jnp.float32),
        mesh=mesh,
        scratch_types=[
            pltpu.VMEM((nt, _CHUNK), jnp.int32),
            pltpu.VMEM((_CHUNK,), jnp.float32),
            pltpu.VMEM((rows_sub,), jnp.float32),
            [pltpu.SemaphoreType.DMA for _ in range(4)],
            pltpu.VMEM_SHARED((np_rows,), jnp.float32),
        ],
    )
    def deg_kernel(dst_hbm, out_hbm, idx_v, ones_v, zero_v, sems, acc_sh):
        cid = lax.axis_index("c")
        sid = lax.axis_index("s")
        z16 = jnp.zeros((16,), jnp.float32)
        o16 = jnp.ones((16,), jnp.float32)

        def zloop(i, _):
            zero_v[pl.ds(i * 16, 16)] = z16
            return 0

        lax.fori_loop(0, rows_sub // 16, zloop, 0)

        def oloop(i, _):
            ones_v[pl.ds(i * 16, 16)] = o16
            return 0

        lax.fori_loop(0, _CHUNK // 16, oloop, 0)
        pltpu.sync_copy(zero_v, acc_sh.at[pl.ds(sid * rows_sub, rows_sub)])
        plsc.subcore_barrier()
        pltpu.sync_copy(dst_hbm.at[pl.ds(sid * nt, nt)], idx_v)

        def chunk(g, _):
            j0 = g * 4
            for b in range(4):
                pltpu.async_copy(ones_v, acc_sh.at[idx_v.at[j0 + b]],
                                 sems[b], add=True)
            for b in range(4):
                pltpu.make_async_copy(ones_v, acc_sh.at[idx_v.at[0]],
                                      sems[b]).wait()
            return 0

        lax.fori_loop(0, nt // 4, chunk, 0)
        plsc.subcore_barrier()
        pltpu.sync_copy(
            acc_sh.at[pl.ds(sid * rows_sub, rows_sub)],
            out_hbm.at[pl.ds(cid * np_rows + sid * rows_sub, rows_sub)],
        )

    return deg_kernel


def _make_scatter_kernel(hdim: int, nchunks: int, np_rows: int):
    mesh = plsc.VectorSubcoreMesh(core_axis_name="c", subcore_axis_name="s")
    nt = nchunks // (_NS * _NC)
    nblk = nt // _IBLK
    rows_sub = np_rows // _NS
    bq, br = divmod(nblk, _NPH)
    phases = [bq + (i < br) for i in range(_NPH)]
    nblk_max = max(phases)

    @functools.partial(
        pl.kernel,
        out_type=jax.ShapeDtypeStruct((_NC, np_rows, hdim), jnp.float32),
        mesh=mesh,
        scratch_types=[
            pltpu.VMEM((nblk_max, _IBLK, _CHUNK), jnp.int32),
            pltpu.VMEM((nblk_max, _IBLK, _CHUNK), jnp.int32),
            [pltpu.VMEM((_CHUNK, hdim), jnp.float32) for _ in range(_NBUF)],
            [pltpu.SemaphoreType.DMA for _ in range(_NBUF)],
            [pltpu.SemaphoreType.DMA for _ in range(_NBUF)],
            pltpu.VMEM_SHARED((np_rows, hdim), jnp.float32),
        ],
    )
    def scat_kernel(tab_hbm, src_hbm, dst_hbm, out_hbm,
                    sidx, didx, bufs, gsems, ssems, acc_sh):
        cid = lax.axis_index("c")
        sid = lax.axis_index("s")
        gid = sid * _NC + cid
        z16 = jnp.zeros((16,), jnp.float32)
        buf0 = bufs[0]

        blk00 = gid * nblk
        pltpu.async_copy(src_hbm.at[pl.ds(blk00, phases[0])],
                         sidx.at[pl.ds(0, phases[0])], gsems[0])
        pltpu.async_copy(dst_hbm.at[pl.ds(blk00, phases[0])],
                         didx.at[pl.ds(0, phases[0])], gsems[1])

        def zrow(r, _):
            for k in range(hdim // 16):
                buf0[r, pl.ds(k * 16, 16)] = z16
            return 0

        lax.fori_loop(0, _CHUNK, zrow, 0)
        base = sid * rows_sub
        off = 0
        while off < rows_sub:
            step = min(_CHUNK, rows_sub - off)
            pltpu.sync_copy(buf0.at[pl.ds(0, step)],
                            acc_sh.at[pl.ds(base + off, step)])
            off += step
        pltpu.make_async_copy(src_hbm.at[pl.ds(blk00, phases[0])],
                              sidx.at[pl.ds(0, phases[0])], gsems[0]).wait()
        pltpu.make_async_copy(dst_hbm.at[pl.ds(blk00, phases[0])],
                              didx.at[pl.ds(0, phases[0])], gsems[1]).wait()
        plsc.subcore_barrier()

        def gfire(j, b):
            pltpu.async_copy(tab_hbm.at[sidx.at[j // _IBLK, j % _IBLK]],
                             bufs[b], gsems[b])

        def gwait(b):
            pltpu.make_async_copy(tab_hbm.at[sidx.at[0, 0]],
                                  bufs[b], gsems[b]).wait()

        def sfire(j, b):
            pltpu.async_copy(bufs[b], acc_sh.at[didx.at[j // _IBLK, j % _IBLK]],
                             ssems[b], add=True)

        def swait(b):
            pltpu.make_async_copy(bufs[b], acc_sh.at[didx.at[0, 0]],
                                  ssems[b]).wait()

        blk_done = 0
        for ph in range(_NPH):
            nblk_ph = phases[ph]
            ngroups = nblk_ph * _IBLK // _NBUF
            blk0 = gid * nblk + blk_done
            blk_done += nblk_ph
            if ph > 0:
                pltpu.sync_copy(src_hbm.at[pl.ds(blk0, nblk_ph)],
                                sidx.at[pl.ds(0, nblk_ph)])
                pltpu.sync_copy(dst_hbm.at[pl.ds(blk0, nblk_ph)],
                                didx.at[pl.ds(0, nblk_ph)])

            for b in range(_NBUF):
                gfire(b, b)

            def group(g, _):
                j0 = g * _NBUF
                for b in range(_NBUF):
                    gwait(b)
                    sfire(j0 + b, b)

                @pl.when(g < ngroups - 1)
                def _():
                    for b in range(_NBUF):
                        swait(b)
                        gfire(j0 + _NBUF + b, b)
                return 0

            lax.fori_loop(0, ngroups, group, 0)
            for b in range(_NBUF):
                swait(b)

        plsc.subcore_barrier()
        pltpu.sync_copy(
            acc_sh.at[pl.ds(base, rows_sub)],
            out_hbm.at[cid, pl.ds(base, rows_sub)],
        )

    return scat_kernel


def _tc1_body(x_ref, w_ref, degp_ref, o_ref):
    dinv = lax.rsqrt((degp_ref[0] + degp_ref[1]) * 0.5)
    xs = x_ref[...] * dinv
    o_ref[...] = jnp.dot(xs, w_ref[...], preferred_element_type=jnp.float32,
                   precision=lax.Precision.HIGHEST)


def _tc2_body(p_ref, degp_ref, w_ref, b_ref, o_ref):
    dinv = lax.rsqrt((degp_ref[0] + degp_ref[1]) * 0.5)
    m = p_ref[0] + p_ref[1]
    z = jnp.maximum(m * dinv + b_ref[...], 0.0) * dinv
    o_ref[...] = jnp.dot(z, w_ref[...], preferred_element_type=jnp.float32,
                   precision=lax.Precision.HIGHEST)


def _tc3_body(p_ref, degp_ref, b_ref, wfc_ref, bfc_ref, o_ref):
    dinv = lax.rsqrt((degp_ref[0] + degp_ref[1]) * 0.5)
    m = p_ref[0] + p_ref[1]
    z = jnp.maximum(m * dinv + b_ref[...], 0.0)
    o_ref[...] = jnp.dot(z, wfc_ref[...], preferred_element_type=jnp.float32,
                   precision=lax.Precision.HIGHEST) + bfc_ref[...]


def _tc1_call(x, w1f, d3, n, d, hdim, interpret=False):
    grid = (pl.cdiv(n, _BM),)
    return pl.pallas_call(
        _tc1_body,
        grid=grid,
        in_specs=[
            pl.BlockSpec((_BM, d), lambda i: (i, 0)),
            pl.BlockSpec((d, hdim), lambda i: (0, 0)),
            pl.BlockSpec((_NC, _BM, 1), lambda i: (0, i, 0)),
        ],
        out_specs=pl.BlockSpec((_BM, hdim), lambda i: (i, 0)),
        out_shape=jax.ShapeDtypeStruct((n, hdim), jnp.float32),
        interpret=interpret,
    )(x, w1f, d3)


def _tc2_call(p1, d3, w2, b1f, n, hdim, interpret=False):
    grid = (pl.cdiv(n, _BM),)
    return pl.pallas_call(
        _tc2_body,
        grid=grid,
        in_specs=[
            pl.BlockSpec((_NC, _BM, hdim), lambda i: (0, i, 0)),
            pl.BlockSpec((_NC, _BM, 1), lambda i: (0, i, 0)),
            pl.BlockSpec((hdim, hdim), lambda i: (0, 0)),
            pl.BlockSpec((1, hdim), lambda i: (0, 0)),
        ],
        out_specs=pl.BlockSpec((_BM, hdim), lambda i: (i, 0)),
        out_shape=jax.ShapeDtypeStruct((n, hdim), jnp.float32),
        interpret=interpret,
    )(p1, d3, w2, b1f)


def _tc3_call(p2, d3, b2, wfc, bfc, n, hdim, interpret=False):
    grid = (pl.cdiv(n, _BM),)
    return pl.pallas_call(
        _tc3_body,
        grid=grid,
        in_specs=[
            pl.BlockSpec((_NC, _BM, hdim), lambda i: (0, i, 0)),
            pl.BlockSpec((_NC, _BM, 1), lambda i: (0, i, 0)),
            pl.BlockSpec((1, hdim), lambda i: (0, 0)),
            pl.BlockSpec((hdim, 1), lambda i: (0, 0)),
            pl.BlockSpec((1, 1), lambda i: (0, 0)),
        ],
        out_specs=pl.BlockSpec((_BM, 1), lambda i: (i, 0)),
        out_shape=jax.ShapeDtypeStruct((n, 1), jnp.float32),
        interpret=interpret,
    )(p2, d3, b2, wfc, bfc)


def kernel(x, edge_index, W1, b1, gamma, beta, run_mean, run_var, W2, b2, Wfc, bfc):
    n, d = x.shape
    hdim = W1.shape[1]
    e = edge_index.shape[1]

    ep = _round_up(e + n, _NS * _NC * _CHUNK * _IBLK)
    nchunks = ep // _CHUNK
    np_rows = _round_up(n + 1, _NS * _CHUNK)
    pad = ep - e - n
    loop = jnp.arange(n, dtype=jnp.int32)
    pad_src = jnp.arange(pad, dtype=jnp.int32) % n
    pad_dst = n + jnp.arange(pad, dtype=jnp.int32) % (np_rows - n)
    src = jnp.concatenate([edge_index[0], loop, pad_src])
    dst = jnp.concatenate([edge_index[1], loop, pad_dst])
    srcp = src.reshape(nchunks // _IBLK, _IBLK, _CHUNK)
    dstp3 = dst.reshape(nchunks // _IBLK, _IBLK, _CHUNK)
    dstp = dst.reshape(nchunks, _CHUNK)

    s = gamma * lax.rsqrt(run_var + _EPS)
    w1f = W1 * s[None, :]
    b1f = (b1 * s + (beta - run_mean * s)).reshape(1, hdim)
    b2r = b2.reshape(1, hdim)
    bfcr = bfc.reshape(1, 1)

    deg_fn = _make_deg_kernel(nchunks, np_rows)
    scat_fn = _make_scatter_kernel(hdim, nchunks, np_rows)

    degp = deg_fn(dstp)
    d3 = degp.reshape(_NC, np_rows, 1)

    h1 = _tc1_call(x, w1f, d3, n, d, hdim)
    p1 = scat_fn(h1, srcp, dstp3)
    h2 = _tc2_call(p1, d3, W2, b1f, n, hdim)
    p2 = scat_fn(h2, srcp, dstp3)
    preds = _tc3_call(p2, d3, b2r, Wfc, bfcr, n, hdim)
    return preds[:, 0]


# speedup vs baseline: 1.0051x; 1.0051x over previous
import functools

import jax
import jax.numpy as jnp
from jax import lax
from jax.experimental import pallas as pl
from jax.experimental.pallas import tpu as pltpu
from jax.experimental.pallas import tpu_sc as plsc

_EPS = 1e-5
_NC = 2
_NS = 16
_CHUNK = 128
_BM = 512
_NBUF = 2
_NPH = 2
_IBLK = 8


def _round_up(a: int, b: int) -> int:
    return (a + b - 1) // b * b


def _make_deg_kernel(nchunks: int, np_rows: int):
    mesh = plsc.VectorSubcoreMesh(core_axis_name="c", subcore_axis_name="s")
    nt = nchunks // _NS
    rows_sub = np_rows // _NS

    @functools.partial(
        pl.kernel,
        out_type=jax.ShapeDtypeStruct((_NC * np_rows,), jnp.float32),
        mesh=mesh,
        scratch_types=[
            pltpu.VMEM((nt, _CHUNK), jnp.int32),
            pltpu.VMEM((_CHUNK,), jnp.float32),
            pltpu.VMEM((rows_sub,), jnp.float32),
            [pltpu.SemaphoreType.DMA for _ in range(4)],
            pltpu.VMEM_SHARED((np_rows,), jnp.float32),
        ],
    )
    def deg_kernel(dst_hbm, out_hbm, idx_v, ones_v, zero_v, sems, acc_sh):
        cid = lax.axis_index("c")
        sid = lax.axis_index("s")
        z16 = jnp.zeros((16,), jnp.float32)
        o16 = jnp.ones((16,), jnp.float32)

        def zloop(i, _):
            zero_v[pl.ds(i * 16, 16)] = z16
            return 0

        lax.fori_loop(0, rows_sub // 16, zloop, 0)

        def oloop(i, _):
            ones_v[pl.ds(i * 16, 16)] = o16
            return 0

        lax.fori_loop(0, _CHUNK // 16, oloop, 0)
        pltpu.sync_copy(zero_v, acc_sh.at[pl.ds(sid * rows_sub, rows_sub)])
        plsc.subcore_barrier()
        pltpu.sync_copy(dst_hbm.at[pl.ds(sid * nt, nt)], idx_v)

        def chunk(g, _):
            j0 = g * 4
            for b in range(4):
                pltpu.async_copy(ones_v, acc_sh.at[idx_v.at[j0 + b]],
                                 sems[b], add=True)
            for b in range(4):
                pltpu.make_async_copy(ones_v, acc_sh.at[idx_v.at[0]],
                                      sems[b]).wait()
            return 0

        lax.fori_loop(0, nt // 4, chunk, 0)
        plsc.subcore_barrier()
        pltpu.sync_copy(
            acc_sh.at[pl.ds(sid * rows_sub, rows_sub)],
            out_hbm.at[pl.ds(cid * np_rows + sid * rows_sub, rows_sub)],
        )

    return deg_kernel


def _make_scatter_kernel(hdim: int, nchunks: int, np_rows: int):
    mesh = plsc.VectorSubcoreMesh(core_axis_name="c", subcore_axis_name="s")
    nt = nchunks // (_NS * _NC)
    nblk = nt // _IBLK
    rows_sub = np_rows // _NS
    bq, br = divmod(nblk, _NPH)
    phases = [bq + (i < br) for i in range(_NPH)]
    nblk_max = max(phases)

    @functools.partial(
        pl.kernel,
        out_type=jax.ShapeDtypeStruct((_NC, np_rows, hdim), jnp.float32),
        mesh=mesh,
        scratch_types=[
            pltpu.VMEM((nblk_max, _IBLK, _CHUNK), jnp.int32),
            pltpu.VMEM((nblk_max, _IBLK, _CHUNK), jnp.int32),
            [pltpu.VMEM((_CHUNK, hdim), jnp.float32) for _ in range(_NBUF)],
            [pltpu.SemaphoreType.DMA for _ in range(_NBUF)],
            [pltpu.SemaphoreType.DMA for _ in range(_NBUF)],
            pltpu.VMEM_SHARED((np_rows, hdim), jnp.float32),
        ],
    )
    def scat_kernel(tab_hbm, src_hbm, dst_hbm, out_hbm,
                    sidx, didx, bufs, gsems, ssems, acc_sh):
        cid = lax.axis_index("c")
        sid = lax.axis_index("s")
        gid = sid * _NC + cid
        z16 = jnp.zeros((16,), jnp.float32)
        buf0 = bufs[0]

        blk00 = gid * nblk
        pltpu.async_copy(src_hbm.at[pl.ds(blk00, phases[0])],
                         sidx.at[pl.ds(0, phases[0])], gsems[0])
        pltpu.async_copy(dst_hbm.at[pl.ds(blk00, phases[0])],
                         didx.at[pl.ds(0, phases[0])], gsems[1])

        def zrow(r, _):
            for k in range(hdim // 16):
                buf0[r, pl.ds(k * 16, 16)] = z16
            return 0

        lax.fori_loop(0, _CHUNK, zrow, 0)
        base = sid * rows_sub
        off = 0
        while off < rows_sub:
            step = min(_CHUNK, rows_sub - off)
            pltpu.sync_copy(buf0.at[pl.ds(0, step)],
                            acc_sh.at[pl.ds(base + off, step)])
            off += step
        pltpu.make_async_copy(src_hbm.at[pl.ds(blk00, phases[0])],
                              sidx.at[pl.ds(0, phases[0])], gsems[0]).wait()
        pltpu.make_async_copy(dst_hbm.at[pl.ds(blk00, phases[0])],
                              didx.at[pl.ds(0, phases[0])], gsems[1]).wait()
        plsc.subcore_barrier()

        def gfire(j, b):
            pltpu.async_copy(tab_hbm.at[sidx.at[j // _IBLK, j % _IBLK]],
                             bufs[b], gsems[b])

        def gwait(b):
            pltpu.make_async_copy(tab_hbm.at[sidx.at[0, 0]],
                                  bufs[b], gsems[b]).wait()

        def sfire(j, b):
            pltpu.async_copy(bufs[b], acc_sh.at[didx.at[j // _IBLK, j % _IBLK]],
                             ssems[b], add=True)

        def swait(b):
            pltpu.make_async_copy(bufs[b], acc_sh.at[didx.at[0, 0]],
                                  ssems[b]).wait()

        blk_done = 0
        for ph in range(_NPH):
            nblk_ph = phases[ph]
            ngroups = nblk_ph * _IBLK // _NBUF
            blk0 = gid * nblk + blk_done
            blk_done += nblk_ph
            if ph > 0:
                pltpu.sync_copy(src_hbm.at[pl.ds(blk0, nblk_ph)],
                                sidx.at[pl.ds(0, nblk_ph)])
                pltpu.sync_copy(dst_hbm.at[pl.ds(blk0, nblk_ph)],
                                didx.at[pl.ds(0, nblk_ph)])

            for b in range(_NBUF):
                gfire(b, b)

            def group(g, _):
                j0 = g * _NBUF
                for b in range(_NBUF):
                    gwait(b)
                    sfire(j0 + b, b)

                @pl.when(g < ngroups - 1)
                def _():
                    for b in range(_NBUF):
                        swait(b)
                        gfire(j0 + _NBUF + b, b)
                return 0

            lax.fori_loop(0, ngroups, group, 0)
            for b in range(_NBUF):
                swait(b)

        plsc.subcore_barrier()
        pltpu.sync_copy(
            acc_sh.at[pl.ds(base, rows_sub)],
            out_hbm.at[cid, pl.ds(base, rows_sub)],
        )

    return scat_kernel


def _tc1_body(x_ref, w_ref, degp_ref, o_ref):
    dinv = lax.rsqrt((degp_ref[0] + degp_ref[1]) * 0.5)
    xs = x_ref[...] * dinv
    o_ref[...] = jnp.dot(xs, w_ref[...], preferred_element_type=jnp.float32)


def _tc2_body(p_ref, degp_ref, w_ref, b_ref, o_ref):
    dinv = lax.rsqrt((degp_ref[0] + degp_ref[1]) * 0.5)
    m = p_ref[0] + p_ref[1]
    z = jnp.maximum(m * dinv + b_ref[...], 0.0) * dinv
    o_ref[...] = jnp.dot(z, w_ref[...], preferred_element_type=jnp.float32)


def _tc3_body(p_ref, degp_ref, b_ref, wfc_ref, bfc_ref, o_ref):
    dinv = lax.rsqrt((degp_ref[0] + degp_ref[1]) * 0.5)
    m = p_ref[0] + p_ref[1]
    z = jnp.maximum(m * dinv + b_ref[...], 0.0)
    o_ref[...] = jnp.dot(z, wfc_ref[...], preferred_element_type=jnp.float32) + bfc_ref[...]


def _tc1_call(x, w1f, d3, n, d, hdim, interpret=False):
    grid = (pl.cdiv(n, _BM),)
    return pl.pallas_call(
        _tc1_body,
        grid=grid,
        in_specs=[
            pl.BlockSpec((_BM, d), lambda i: (i, 0)),
            pl.BlockSpec((d, hdim), lambda i: (0, 0)),
            pl.BlockSpec((_NC, _BM, 1), lambda i: (0, i, 0)),
        ],
        out_specs=pl.BlockSpec((_BM, hdim), lambda i: (i, 0)),
        out_shape=jax.ShapeDtypeStruct((n, hdim), jnp.float32),
        interpret=interpret,
    )(x, w1f, d3)


def _tc2_call(p1, d3, w2, b1f, n, hdim, interpret=False):
    grid = (pl.cdiv(n, _BM),)
    return pl.pallas_call(
        _tc2_body,
        grid=grid,
        in_specs=[
            pl.BlockSpec((_NC, _BM, hdim), lambda i: (0, i, 0)),
            pl.BlockSpec((_NC, _BM, 1), lambda i: (0, i, 0)),
            pl.BlockSpec((hdim, hdim), lambda i: (0, 0)),
            pl.BlockSpec((1, hdim), lambda i: (0, 0)),
        ],
        out_specs=pl.BlockSpec((_BM, hdim), lambda i: (i, 0)),
        out_shape=jax.ShapeDtypeStruct((n, hdim), jnp.float32),
        interpret=interpret,
    )(p1, d3, w2, b1f)


def _tc3_call(p2, d3, b2, wfc, bfc, n, hdim, interpret=False):
    grid = (pl.cdiv(n, _BM),)
    return pl.pallas_call(
        _tc3_body,
        grid=grid,
        in_specs=[
            pl.BlockSpec((_NC, _BM, hdim), lambda i: (0, i, 0)),
            pl.BlockSpec((_NC, _BM, 1), lambda i: (0, i, 0)),
            pl.BlockSpec((1, hdim), lambda i: (0, 0)),
            pl.BlockSpec((hdim, 1), lambda i: (0, 0)),
            pl.BlockSpec((1, 1), lambda i: (0, 0)),
        ],
        out_specs=pl.BlockSpec((_BM, 1), lambda i: (i, 0)),
        out_shape=jax.ShapeDtypeStruct((n, 1), jnp.float32),
        interpret=interpret,
    )(p2, d3, b2, wfc, bfc)


def kernel(x, edge_index, W1, b1, gamma, beta, run_mean, run_var, W2, b2, Wfc, bfc):
    n, d = x.shape
    hdim = W1.shape[1]
    e = edge_index.shape[1]

    ep = _round_up(e + n, _NS * _NC * _CHUNK * _IBLK)
    nchunks = ep // _CHUNK
    np_rows = _round_up(n + 1, _NS * _CHUNK)
    pad = ep - e - n
    loop = jnp.arange(n, dtype=jnp.int32)
    pad_src = jnp.arange(pad, dtype=jnp.int32) % n
    pad_dst = n + jnp.arange(pad, dtype=jnp.int32) % (np_rows - n)
    src = jnp.concatenate([edge_index[0], loop, pad_src])
    dst = jnp.concatenate([edge_index[1], loop, pad_dst])
    srcp = src.reshape(nchunks // _IBLK, _IBLK, _CHUNK)
    dstp3 = dst.reshape(nchunks // _IBLK, _IBLK, _CHUNK)
    dstp = dst.reshape(nchunks, _CHUNK)

    s = gamma * lax.rsqrt(run_var + _EPS)
    w1f = W1 * s[None, :]
    b1f = (b1 * s + (beta - run_mean * s)).reshape(1, hdim)
    b2r = b2.reshape(1, hdim)
    bfcr = bfc.reshape(1, 1)

    deg_fn = _make_deg_kernel(nchunks, np_rows)
    scat_fn = _make_scatter_kernel(hdim, nchunks, np_rows)

    degp = deg_fn(dstp)
    d3 = degp.reshape(_NC, np_rows, 1)

    h1 = _tc1_call(x, w1f, d3, n, d, hdim)
    p1 = scat_fn(h1, srcp, dstp3)
    h2 = _tc2_call(p1, d3, W2, b1f, n, hdim)
    p2 = scat_fn(h2, srcp, dstp3)
    preds = _tc3_call(p2, d3, b2r, Wfc, bfcr, n, hdim)
    return preds[:, 0]


# speedup vs baseline: 1.0673x; 1.0619x over previous
import functools

import jax
import jax.numpy as jnp
from jax import lax
from jax.experimental import pallas as pl
from jax.experimental.pallas import tpu as pltpu
from jax.experimental.pallas import tpu_sc as plsc

_EPS = 1e-5
_NC = 2
_NS = 16
_CHUNK = 128
_BM = 2048
_NBUF = 2
_NPH = 2
_IBLK = 8


def _round_up(a: int, b: int) -> int:
    return (a + b - 1) // b * b


def _make_deg_kernel(nchunks: int, np_rows: int):
    mesh = plsc.VectorSubcoreMesh(core_axis_name="c", subcore_axis_name="s")
    nt = nchunks // _NS
    rows_sub = np_rows // _NS

    @functools.partial(
        pl.kernel,
        out_type=jax.ShapeDtypeStruct((_NC * np_rows,), jnp.float32),
        mesh=mesh,
        scratch_types=[
            pltpu.VMEM((nt, _CHUNK), jnp.int32),
            pltpu.VMEM((_CHUNK,), jnp.float32),
            pltpu.VMEM((rows_sub,), jnp.float32),
            [pltpu.SemaphoreType.DMA for _ in range(4)],
            pltpu.VMEM_SHARED((np_rows,), jnp.float32),
        ],
    )
    def deg_kernel(dst_hbm, out_hbm, idx_v, ones_v, zero_v, sems, acc_sh):
        cid = lax.axis_index("c")
        sid = lax.axis_index("s")
        z16 = jnp.zeros((16,), jnp.float32)
        o16 = jnp.ones((16,), jnp.float32)

        def zloop(i, _):
            zero_v[pl.ds(i * 16, 16)] = z16
            return 0

        lax.fori_loop(0, rows_sub // 16, zloop, 0)

        def oloop(i, _):
            ones_v[pl.ds(i * 16, 16)] = o16
            return 0

        lax.fori_loop(0, _CHUNK // 16, oloop, 0)
        pltpu.sync_copy(zero_v, acc_sh.at[pl.ds(sid * rows_sub, rows_sub)])
        plsc.subcore_barrier()
        pltpu.sync_copy(dst_hbm.at[pl.ds(sid * nt, nt)], idx_v)

        def chunk(g, _):
            j0 = g * 4
            for b in range(4):
                pltpu.async_copy(ones_v, acc_sh.at[idx_v.at[j0 + b]],
                                 sems[b], add=True)
            for b in range(4):
                pltpu.make_async_copy(ones_v, acc_sh.at[idx_v.at[0]],
                                      sems[b]).wait()
            return 0

        lax.fori_loop(0, nt // 4, chunk, 0)
        plsc.subcore_barrier()
        pltpu.sync_copy(
            acc_sh.at[pl.ds(sid * rows_sub, rows_sub)],
            out_hbm.at[pl.ds(cid * np_rows + sid * rows_sub, rows_sub)],
        )

    return deg_kernel


def _make_scatter_kernel(hdim: int, nchunks: int, np_rows: int):
    mesh = plsc.VectorSubcoreMesh(core_axis_name="c", subcore_axis_name="s")
    nt = nchunks // (_NS * _NC)
    nblk = nt // _IBLK
    rows_sub = np_rows // _NS
    bq, br = divmod(nblk, _NPH)
    phases = [bq + (i < br) for i in range(_NPH)]
    nblk_max = max(phases)

    @functools.partial(
        pl.kernel,
        out_type=jax.ShapeDtypeStruct((_NC, np_rows, hdim), jnp.float32),
        mesh=mesh,
        scratch_types=[
            pltpu.VMEM((nblk_max, _IBLK, _CHUNK), jnp.int32),
            pltpu.VMEM((nblk_max, _IBLK, _CHUNK), jnp.int32),
            [pltpu.VMEM((_CHUNK, hdim), jnp.float32) for _ in range(_NBUF)],
            [pltpu.SemaphoreType.DMA for _ in range(_NBUF)],
            [pltpu.SemaphoreType.DMA for _ in range(_NBUF)],
            pltpu.VMEM_SHARED((np_rows, hdim), jnp.float32),
        ],
    )
    def scat_kernel(tab_hbm, src_hbm, dst_hbm, out_hbm,
                    sidx, didx, bufs, gsems, ssems, acc_sh):
        cid = lax.axis_index("c")
        sid = lax.axis_index("s")
        gid = sid * _NC + cid
        z16 = jnp.zeros((16,), jnp.float32)
        buf0 = bufs[0]

        blk00 = gid * nblk
        pltpu.async_copy(src_hbm.at[pl.ds(blk00, phases[0])],
                         sidx.at[pl.ds(0, phases[0])], gsems[0])
        pltpu.async_copy(dst_hbm.at[pl.ds(blk00, phases[0])],
                         didx.at[pl.ds(0, phases[0])], gsems[1])

        def zrow(r, _):
            for k in range(hdim // 16):
                buf0[r, pl.ds(k * 16, 16)] = z16
            return 0

        lax.fori_loop(0, _CHUNK, zrow, 0)
        base = sid * rows_sub
        off = 0
        while off < rows_sub:
            step = min(_CHUNK, rows_sub - off)
            pltpu.sync_copy(buf0.at[pl.ds(0, step)],
                            acc_sh.at[pl.ds(base + off, step)])
            off += step
        pltpu.make_async_copy(src_hbm.at[pl.ds(blk00, phases[0])],
                              sidx.at[pl.ds(0, phases[0])], gsems[0]).wait()
        pltpu.make_async_copy(dst_hbm.at[pl.ds(blk00, phases[0])],
                              didx.at[pl.ds(0, phases[0])], gsems[1]).wait()
        plsc.subcore_barrier()

        def gfire(j, b):
            pltpu.async_copy(tab_hbm.at[sidx.at[j // _IBLK, j % _IBLK]],
                             bufs[b], gsems[b])

        def gwait(b):
            pltpu.make_async_copy(tab_hbm.at[sidx.at[0, 0]],
                                  bufs[b], gsems[b]).wait()

        def sfire(j, b):
            pltpu.async_copy(bufs[b], acc_sh.at[didx.at[j // _IBLK, j % _IBLK]],
                             ssems[b], add=True)

        def swait(b):
            pltpu.make_async_copy(bufs[b], acc_sh.at[didx.at[0, 0]],
                                  ssems[b]).wait()

        blk_done = 0
        for ph in range(_NPH):
            nblk_ph = phases[ph]
            ngroups = nblk_ph * _IBLK // _NBUF
            blk0 = gid * nblk + blk_done
            blk_done += nblk_ph
            if ph > 0:
                pltpu.sync_copy(src_hbm.at[pl.ds(blk0, nblk_ph)],
                                sidx.at[pl.ds(0, nblk_ph)])
                pltpu.sync_copy(dst_hbm.at[pl.ds(blk0, nblk_ph)],
                                didx.at[pl.ds(0, nblk_ph)])

            for b in range(_NBUF):
                gfire(b, b)

            def group(g, _):
                j0 = g * _NBUF
                for b in range(_NBUF):
                    gwait(b)
                    sfire(j0 + b, b)

                @pl.when(g < ngroups - 1)
                def _():
                    for b in range(_NBUF):
                        swait(b)
                        gfire(j0 + _NBUF + b, b)
                return 0

            lax.fori_loop(0, ngroups, group, 0)
            for b in range(_NBUF):
                swait(b)

        plsc.subcore_barrier()
        pltpu.sync_copy(
            acc_sh.at[pl.ds(base, rows_sub)],
            out_hbm.at[cid, pl.ds(base, rows_sub)],
        )

    return scat_kernel


def _tc1_body(x_ref, w_ref, degp_ref, o_ref):
    dinv = lax.rsqrt((degp_ref[0] + degp_ref[1]) * 0.5)
    xs = x_ref[...] * dinv
    o_ref[...] = jnp.dot(xs, w_ref[...], preferred_element_type=jnp.float32)


def _tc2_body(p_ref, degp_ref, w_ref, b_ref, o_ref):
    dinv = lax.rsqrt((degp_ref[0] + degp_ref[1]) * 0.5)
    m = p_ref[0] + p_ref[1]
    z = jnp.maximum(m * dinv + b_ref[...], 0.0) * dinv
    o_ref[...] = jnp.dot(z, w_ref[...], preferred_element_type=jnp.float32)


def _tc3_body(p_ref, degp_ref, b_ref, wfc_ref, bfc_ref, o_ref):
    dinv = lax.rsqrt((degp_ref[0] + degp_ref[1]) * 0.5)
    m = p_ref[0] + p_ref[1]
    z = jnp.maximum(m * dinv + b_ref[...], 0.0)
    o_ref[...] = jnp.dot(z, wfc_ref[...], preferred_element_type=jnp.float32) + bfc_ref[...]


def _tc1_call(x, w1f, d3, n, d, hdim, interpret=False):
    grid = (pl.cdiv(n, _BM),)
    return pl.pallas_call(
        _tc1_body,
        grid=grid,
        in_specs=[
            pl.BlockSpec((_BM, d), lambda i: (i, 0)),
            pl.BlockSpec((d, hdim), lambda i: (0, 0)),
            pl.BlockSpec((_NC, _BM, 1), lambda i: (0, i, 0)),
        ],
        out_specs=pl.BlockSpec((_BM, hdim), lambda i: (i, 0)),
        out_shape=jax.ShapeDtypeStruct((n, hdim), jnp.float32),
        interpret=interpret,
    )(x, w1f, d3)


def _tc2_call(p1, d3, w2, b1f, n, hdim, interpret=False):
    grid = (pl.cdiv(n, _BM),)
    return pl.pallas_call(
        _tc2_body,
        grid=grid,
        in_specs=[
            pl.BlockSpec((_NC, _BM, hdim), lambda i: (0, i, 0)),
            pl.BlockSpec((_NC, _BM, 1), lambda i: (0, i, 0)),
            pl.BlockSpec((hdim, hdim), lambda i: (0, 0)),
            pl.BlockSpec((1, hdim), lambda i: (0, 0)),
        ],
        out_specs=pl.BlockSpec((_BM, hdim), lambda i: (i, 0)),
        out_shape=jax.ShapeDtypeStruct((n, hdim), jnp.float32),
        interpret=interpret,
    )(p1, d3, w2, b1f)


def _tc3_call(p2, d3, b2, wfc, bfc, n, hdim, interpret=False):
    grid = (pl.cdiv(n, _BM),)
    return pl.pallas_call(
        _tc3_body,
        grid=grid,
        in_specs=[
            pl.BlockSpec((_NC, _BM, hdim), lambda i: (0, i, 0)),
            pl.BlockSpec((_NC, _BM, 1), lambda i: (0, i, 0)),
            pl.BlockSpec((1, hdim), lambda i: (0, 0)),
            pl.BlockSpec((hdim, 1), lambda i: (0, 0)),
            pl.BlockSpec((1, 1), lambda i: (0, 0)),
        ],
        out_specs=pl.BlockSpec((_BM, 1), lambda i: (i, 0)),
        out_shape=jax.ShapeDtypeStruct((n, 1), jnp.float32),
        interpret=interpret,
    )(p2, d3, b2, wfc, bfc)


def kernel(x, edge_index, W1, b1, gamma, beta, run_mean, run_var, W2, b2, Wfc, bfc):
    n, d = x.shape
    hdim = W1.shape[1]
    e = edge_index.shape[1]

    ep = _round_up(e + n, _NS * _NC * _CHUNK * _IBLK)
    nchunks = ep // _CHUNK
    np_rows = _round_up(n + 1, _NS * _CHUNK)
    pad = ep - e - n
    loop = jnp.arange(n, dtype=jnp.int32)
    pad_src = jnp.arange(pad, dtype=jnp.int32) % n
    pad_dst = n + jnp.arange(pad, dtype=jnp.int32) % (np_rows - n)
    src = jnp.concatenate([edge_index[0], loop, pad_src])
    dst = jnp.concatenate([edge_index[1], loop, pad_dst])
    srcp = src.reshape(nchunks // _IBLK, _IBLK, _CHUNK)
    dstp3 = dst.reshape(nchunks // _IBLK, _IBLK, _CHUNK)
    dstp = dst.reshape(nchunks, _CHUNK)

    s = gamma * lax.rsqrt(run_var + _EPS)
    w1f = W1 * s[None, :]
    b1f = (b1 * s + (beta - run_mean * s)).reshape(1, hdim)
    b2r = b2.reshape(1, hdim)
    bfcr = bfc.reshape(1, 1)

    deg_fn = _make_deg_kernel(nchunks, np_rows)
    scat_fn = _make_scatter_kernel(hdim, nchunks, np_rows)

    degp = deg_fn(dstp)
    d3 = degp.reshape(_NC, np_rows, 1)

    h1 = _tc1_call(x, w1f, d3, n, d, hdim)
    p1 = scat_fn(h1, srcp, dstp3)
    h2 = _tc2_call(p1, d3, W2, b1f, n, hdim)
    p2 = scat_fn(h2, srcp, dstp3)
    preds = _tc3_call(p2, d3, b2r, Wfc, bfcr, n, hdim)
    return preds[:, 0]
